# Initial kernel scaffold; baseline (speedup 1.0000x reference)
import jax
import jax.numpy as jnp
from jax.experimental import pallas as pl


def kernel(x, conv_w, conv_b, cls_w, cls_b, bbox_w, bbox_b):
    raise NotImplementedError("write your pallas kernel here")


# speedup vs baseline: 5.3033x; 5.3033x over previous
import jax
import jax.numpy as jnp
from jax.experimental import pallas as pl

TH = 16


def _rpn_block_kernel(x_ref, top_ref, bot_ref, a_ref, wh_ref, cb_ref, hb_ref,
                      logits_ref, bbox_ref):
    cw = jnp.bfloat16
    xb = x_ref[0].astype(cw)
    top = top_ref[0, 0].astype(cw)
    bot = bot_ref[0, 0].astype(cw)
    a = a_ref[...]
    wh = wh_ref[...]
    cb = cb_ref[...]
    hb = hb_ref[...]

    c, th, w = xb.shape
    rows = [xb[:, j, :] for j in range(th)]
    zc = jnp.zeros((c, 1), jnp.float32)

    for h in range(th):
        r0 = top if h == 0 else rows[h - 1]
        r2 = bot if h == th - 1 else rows[h + 1]
        r = jnp.concatenate([r0, rows[h], r2], axis=0)
        y = jax.lax.dot_general(a, r, (((1,), (0,)), ((), ())),
                                preferred_element_type=jnp.float32)
        acc = (y[c:2 * c, :]
               + jnp.concatenate([zc, y[:c, :-1]], axis=1)
               + jnp.concatenate([y[2 * c:, 1:], zc], axis=1)
               + cb)
        t = jnp.maximum(acc, 0.0).astype(cw)
        hd = jax.lax.dot_general(wh, t, (((1,), (0,)), ((), ())),
                                 preferred_element_type=jnp.float32) + hb
        logits_ref[0, :, h, :] = hd[0:15, :]
        bbox_ref[0, :, h, :] = hd[15:90, :]


def kernel(x, conv_w, conv_b, cls_w, cls_b, bbox_w, bbox_b):
    n, c, hh, ww = x.shape
    nb = hh // TH

    zrow = jnp.zeros((n, 1, c, ww), x.dtype)
    tops = jnp.transpose(x[:, :, TH - 1::TH, :], (0, 2, 1, 3))[:, :-1]
    tops = jnp.concatenate([zrow, tops], axis=1)
    bots = jnp.transpose(x[:, :, TH::TH, :], (0, 2, 1, 3))
    bots = jnp.concatenate([bots, zrow], axis=1)

    a = jnp.transpose(conv_w, (3, 0, 2, 1)).reshape(3 * c, 3 * c)
    a = a.astype(jnp.bfloat16)
    whead = jnp.concatenate([cls_w[:, :, 0, 0], bbox_w[:, :, 0, 0],
                             jnp.zeros((6, c), jnp.float32)], axis=0)
    whead = whead.astype(jnp.bfloat16)
    cb = conv_b.reshape(c, 1)
    hb = jnp.concatenate([cls_b, bbox_b, jnp.zeros((6,), jnp.float32)])
    hb = hb.reshape(96, 1)

    grid = (n, nb)
    out_shape = (jax.ShapeDtypeStruct((n, 15, hh, ww), jnp.float32),
                 jax.ShapeDtypeStruct((n, 75, hh, ww), jnp.float32))
    return pl.pallas_call(
        _rpn_block_kernel,
        grid=grid,
        in_specs=[
            pl.BlockSpec((1, c, TH, ww), lambda bn, bi: (bn, 0, bi, 0)),
            pl.BlockSpec((1, 1, c, ww), lambda bn, bi: (bn, bi, 0, 0)),
            pl.BlockSpec((1, 1, c, ww), lambda bn, bi: (bn, bi, 0, 0)),
            pl.BlockSpec((3 * c, 3 * c), lambda bn, bi: (0, 0)),
            pl.BlockSpec((96, c), lambda bn, bi: (0, 0)),
            pl.BlockSpec((c, 1), lambda bn, bi: (0, 0)),
            pl.BlockSpec((96, 1), lambda bn, bi: (0, 0)),
        ],
        out_specs=(
            pl.BlockSpec((1, 15, TH, ww), lambda bn, bi: (bn, 0, bi, 0)),
            pl.BlockSpec((1, 75, TH, ww), lambda bn, bi: (bn, 0, bi, 0)),
        ),
        out_shape=out_shape,
    )(x, tops, bots, a, whead, cb, hb)


# speedup vs baseline: 10.4116x; 1.9632x over previous
import jax
import jax.numpy as jnp
from jax.experimental import pallas as pl

TH = 16


def _rpn_block_kernel(x_ref, top_ref, bot_ref, a_ref, wh_ref, cb_ref, hb_ref,
                      logits_ref, bbox_ref):
    cw = jnp.bfloat16
    a = a_ref[...]
    wh = wh_ref[...]
    cb = cb_ref[...]
    hb = hb_ref[...]

    _, th, c, w = x_ref.shape
    bi = pl.program_id(1)
    nb = pl.num_programs(1)
    zrow = jnp.zeros((c, w), cw)
    rows = [None] * (th + 2)
    rows[0] = jnp.where(bi == 0, zrow, top_ref[0, 0])
    for j in range(th):
        rows[j + 1] = x_ref[0, j]
    rows[th + 1] = jnp.where(bi == nb - 1, zrow, bot_ref[0, 0])
    zc = jnp.zeros((c, 1), jnp.float32)

    dn = (((1,), (0,)), ((), ()))
    JAM = 8
    for h0 in range(0, th, JAM):
        hs = range(h0, h0 + JAM)
        rs = [jnp.concatenate([rows[h], rows[h + 1], rows[h + 2]], axis=0)
              for h in hs]
        ys = [jax.lax.dot_general(a, r, dn, preferred_element_type=jnp.float32)
              for r in rs]
        ts = []
        for y in ys:
            acc = (y[c:2 * c, :]
                   + jnp.concatenate([zc, y[:c, :-1]], axis=1)
                   + jnp.concatenate([y[2 * c:, 1:], zc], axis=1)
                   + cb)
            ts.append(jnp.maximum(acc, 0.0).astype(cw))
        hds = [jax.lax.dot_general(wh, t, dn, preferred_element_type=jnp.float32)
               + hb for t in ts]
        for h, hd in zip(hs, hds):
            logits_ref[0, :, h, :] = hd[0:15, :]
            bbox_ref[0, :, h, :] = hd[15:90, :]


def kernel(x, conv_w, conv_b, cls_w, cls_b, bbox_w, bbox_b):
    n, c, hh, ww = x.shape
    nb = hh // TH

    xt = jnp.transpose(x, (0, 2, 1, 3)).astype(jnp.bfloat16)

    a = jnp.transpose(conv_w, (3, 0, 2, 1)).reshape(3 * c, 3 * c)
    a = a.astype(jnp.bfloat16)
    whead = jnp.concatenate([cls_w[:, :, 0, 0], bbox_w[:, :, 0, 0],
                             jnp.zeros((6, c), jnp.float32)], axis=0)
    whead = whead.astype(jnp.bfloat16)
    cb = conv_b.reshape(c, 1)
    hb = jnp.concatenate([cls_b, bbox_b, jnp.zeros((6,), jnp.float32)])
    hb = hb.reshape(96, 1)

    out_shape = (jax.ShapeDtypeStruct((n, 15, hh, ww), jnp.float32),
                 jax.ShapeDtypeStruct((n, 75, hh, ww), jnp.float32))
    return pl.pallas_call(
        _rpn_block_kernel,
        grid=(n, nb),
        in_specs=[
            pl.BlockSpec((1, TH, c, ww), lambda bn, bi: (bn, bi, 0, 0)),
            pl.BlockSpec((1, 1, c, ww),
                         lambda bn, bi: (bn, jnp.maximum(bi * TH - 1, 0), 0, 0)),
            pl.BlockSpec((1, 1, c, ww),
                         lambda bn, bi: (bn, jnp.minimum(bi * TH + TH, hh - 1), 0, 0)),
            pl.BlockSpec((3 * c, 3 * c), lambda bn, bi: (0, 0)),
            pl.BlockSpec((96, c), lambda bn, bi: (0, 0)),
            pl.BlockSpec((c, 1), lambda bn, bi: (0, 0)),
            pl.BlockSpec((96, 1), lambda bn, bi: (0, 0)),
        ],
        out_specs=(
            pl.BlockSpec((1, 15, TH, ww), lambda bn, bi: (bn, 0, bi, 0)),
            pl.BlockSpec((1, 75, TH, ww), lambda bn, bi: (bn, 0, bi, 0)),
        ),
        out_shape=out_shape,
    )(xt, xt, xt, a, whead, cb, hb)
